# Initial kernel scaffold; baseline (speedup 1.0000x reference)
import functools

import jax
import jax.numpy as jnp
from jax import lax
from jax.experimental import pallas as pl
from jax.experimental.pallas import tpu as pltpu
from jax.experimental.pallas import tpu_sc as plsc


def _concat_tables(old2, new2, n_old_blk, n_new_blk, blk):
    def body(old_ref, new_ref, out_ref):
        i = pl.program_id(0)

        @pl.when(i < n_old_blk)
        def _():
            out_ref[...] = old_ref[...]

        @pl.when(i >= n_old_blk)
        def _():
            out_ref[...] = new_ref[...]

    total = n_old_blk + n_new_blk
    return pl.pallas_call(
        body,
        grid=(total,),
        in_specs=[
            pl.BlockSpec((blk, 128), lambda i: (jnp.minimum(i, n_old_blk - 1), 0)),
            pl.BlockSpec((blk, 128), lambda i: (jnp.maximum(i - n_old_blk, 0), 0)),
        ],
        out_specs=pl.BlockSpec((blk, 128), lambda i: (i, 0)),
        out_shape=jax.ShapeDtypeStruct((total * blk, 128), jnp.float32),
    )(old2, new2)


def kernel(input_ids, old_weight, new_weight):
    old_vocab, d = old_weight.shape
    new_vocab = new_weight.shape[0]
    batch, hist = input_ids.shape
    n = batch * hist

    ids = input_ids.reshape(n).astype(jnp.int32)

    packf = 128 // d
    blk = 500
    n_old_blk = old_vocab // packf // blk
    n_new_blk = new_vocab // packf // blk
    combined = _concat_tables(
        old_weight.reshape(old_vocab // packf, 128),
        new_weight.reshape(new_vocab // packf, 128),
        n_old_blk,
        n_new_blk,
        blk,
    ).reshape(old_vocab + new_vocab, d)

    info = plsc.get_sparse_core_info()
    nc, ns = info.num_cores, info.num_subcores
    nw = nc * ns
    per_w = n // nw
    sub = 128
    ch = 512
    iters = per_w // ch

    mesh = plsc.VectorSubcoreMesh(core_axis_name="c", subcore_axis_name="s")

    @functools.partial(
        pl.kernel,
        mesh=mesh,
        out_type=jax.ShapeDtypeStruct((n, d), jnp.float32),
        scratch_types=[
            pltpu.VMEM((ch,), jnp.int32),
            pltpu.VMEM((ch, d), jnp.float32),
            pltpu.SemaphoreType.DMA,
        ],
    )
    def gather_k(tbl_hbm, ids_hbm, out_hbm, idx_v, rows_v, sem):
        wid = lax.axis_index("s") * nc + lax.axis_index("c")
        base = wid * per_w

        def body(it, carry):
            off = base + it * ch
            pltpu.sync_copy(ids_hbm.at[pl.ds(off, ch)], idx_v)
            copies = []
            for k in range(ch // sub):
                copies.append(
                    pltpu.async_copy(
                        tbl_hbm.at[idx_v.at[pl.ds(k * sub, sub)]],
                        rows_v.at[pl.ds(k * sub, sub)],
                        sem,
                    )
                )
            for cp in copies:
                cp.wait()
            pltpu.sync_copy(rows_v, out_hbm.at[pl.ds(off, ch)])
            return carry

        lax.fori_loop(0, iters, body, 0)

    out = gather_k(combined, ids)
    return out.reshape(batch, hist, d)


# speedup vs baseline: 7.6722x; 7.6722x over previous
import functools

import jax
import jax.numpy as jnp
from jax import lax
from jax.experimental import pallas as pl
from jax.experimental.pallas import tpu as pltpu
from jax.experimental.pallas import tpu_sc as plsc


def _concat_tables(old2, new2, n_old_blk, n_new_blk, blk):
    def body(old_ref, new_ref, out_ref):
        i = pl.program_id(0)

        @pl.when(i < n_old_blk)
        def _():
            out_ref[...] = old_ref[...]

        @pl.when(i >= n_old_blk)
        def _():
            out_ref[...] = new_ref[...]

    total = n_old_blk + n_new_blk
    return pl.pallas_call(
        body,
        grid=(total,),
        in_specs=[
            pl.BlockSpec((blk, 128), lambda i: (jnp.minimum(i, n_old_blk - 1), 0)),
            pl.BlockSpec((blk, 128), lambda i: (jnp.maximum(i - n_old_blk, 0), 0)),
        ],
        out_specs=pl.BlockSpec((blk, 128), lambda i: (i, 0)),
        out_shape=jax.ShapeDtypeStruct((total * blk, 128), jnp.float32),
    )(old2, new2)


def kernel(input_ids, old_weight, new_weight):
    old_vocab, d = old_weight.shape
    new_vocab = new_weight.shape[0]
    batch, hist = input_ids.shape
    n = batch * hist

    ids = input_ids.reshape(n).astype(jnp.int32)

    packf = 128 // d
    blk = 1000
    n_old_blk = old_vocab // packf // blk
    n_new_blk = new_vocab // packf // blk
    combined = _concat_tables(
        old_weight.reshape(old_vocab // packf, 128),
        new_weight.reshape(new_vocab // packf, 128),
        n_old_blk,
        n_new_blk,
        blk,
    ).reshape(old_vocab + new_vocab, d)

    info = plsc.get_sparse_core_info()
    nc, ns = info.num_cores, info.num_subcores
    nw = nc * ns
    per_w = n // nw
    sub = 128
    ch = 512
    iters = per_w // ch

    mesh = plsc.VectorSubcoreMesh(core_axis_name="c", subcore_axis_name="s")

    @functools.partial(
        pl.kernel,
        mesh=mesh,
        compiler_params=pltpu.CompilerParams(use_tc_tiling_on_sc=False),
        out_type=jax.ShapeDtypeStruct((n, d), jnp.float32),
        scratch_types=[
            pltpu.VMEM((ch,), jnp.int32),
            pltpu.VMEM((ch, d), jnp.float32),
            pltpu.SemaphoreType.DMA,
        ],
    )
    def gather_k(tbl_hbm, ids_hbm, out_hbm, idx_v, rows_v, sem):
        wid = lax.axis_index("s") * nc + lax.axis_index("c")
        base = wid * per_w

        def body(it, carry):
            off = base + it * ch
            pltpu.sync_copy(ids_hbm.at[pl.ds(off, ch)], idx_v)
            copies = []
            for k in range(ch // sub):
                copies.append(
                    pltpu.async_copy(
                        tbl_hbm.at[idx_v.at[pl.ds(k * sub, sub)]],
                        rows_v.at[pl.ds(k * sub, sub)],
                        sem,
                    )
                )
            for cp in copies:
                cp.wait()
            pltpu.sync_copy(rows_v, out_hbm.at[pl.ds(off, ch)])
            return carry

        lax.fori_loop(0, iters, body, 0)

    out = gather_k(combined, ids)
    return out.reshape(batch, hist, d)
